# Initial kernel scaffold; baseline (speedup 1.0000x reference)
import jax
import jax.numpy as jnp
from jax.experimental import pallas as pl


def kernel(x, weight, bias):
    raise NotImplementedError("write your pallas kernel here")


# speedup vs baseline: 1.3780x; 1.3780x over previous
import jax
import jax.numpy as jnp
from jax.experimental import pallas as pl
from jax.experimental.pallas import tpu as pltpu


def _dense_packed_kernel(x_ref, w_ref, b_ref, o_ref):
    o_ref[...] = (
        jnp.dot(x_ref[...], w_ref[...], preferred_element_type=jnp.float32)
        + b_ref[0]
    )


def _rowsum_kernel(x_ref, w_ref, b_ref, o_ref):
    x = x_ref[...].astype(jnp.float32)
    w = w_ref[...].astype(jnp.float32)
    o_ref[...] = jnp.sum(x * w, axis=-1, keepdims=True) + b_ref[0]


def kernel(x, weight, bias):
    B, K = x.shape
    bias_f32 = bias.astype(jnp.float32).reshape(1)

    packable = (
        x.dtype == jnp.float32
        and K < 128
        and 128 % K == 0
        and (B * K) % 4096 == 0
    )

    if packable:
        cols = 4096
        spr = cols // K
        rows = (B * K) // cols
        x_dense = x.reshape(rows, cols)

        w_row = weight.astype(jnp.float32).reshape(K)
        w_big = (
            jnp.eye(spr, dtype=jnp.float32)[:, None, :] * w_row[None, :, None]
        ).reshape(cols, spr)

        tb = max(8, min(512, ((rows + 1) // 2) // 8 * 8))
        grid = (pl.cdiv(rows, tb),)

        out = pl.pallas_call(
            _dense_packed_kernel,
            out_shape=jax.ShapeDtypeStruct((rows, spr), jnp.float32),
            grid_spec=pltpu.PrefetchScalarGridSpec(
                num_scalar_prefetch=0,
                grid=grid,
                in_specs=[
                    pl.BlockSpec((tb, cols), lambda i: (i, 0)),
                    pl.BlockSpec((cols, spr), lambda i: (0, 0)),
                    pl.BlockSpec(memory_space=pltpu.MemorySpace.SMEM),
                ],
                out_specs=pl.BlockSpec((tb, spr), lambda i: (i, 0)),
            ),
            compiler_params=pltpu.CompilerParams(
                dimension_semantics=("parallel",),
                vmem_limit_bytes=64 * 1024 * 1024,
            ),
        )(x_dense, w_big, bias_f32)
        return out.reshape(B, 1)

    tb = max(8, min(4096, ((B + 1) // 2) // 8 * 8))
    grid = (pl.cdiv(B, tb),)
    return pl.pallas_call(
        _rowsum_kernel,
        out_shape=jax.ShapeDtypeStruct((B, 1), jnp.float32),
        grid_spec=pltpu.PrefetchScalarGridSpec(
            num_scalar_prefetch=0,
            grid=grid,
            in_specs=[
                pl.BlockSpec((tb, K), lambda i: (i, 0)),
                pl.BlockSpec((1, K), lambda i: (0, 0)),
                pl.BlockSpec(memory_space=pltpu.MemorySpace.SMEM),
            ],
            out_specs=pl.BlockSpec((tb, 1), lambda i: (i, 0)),
        ),
        compiler_params=pltpu.CompilerParams(
            dimension_semantics=("parallel",),
            vmem_limit_bytes=64 * 1024 * 1024,
        ),
    )(x, weight, bias_f32)
